# baseline (speedup 1.0000x reference)
import functools

import jax
import jax.numpy as jnp
from jax import lax
from jax.experimental import pallas as pl
from jax.experimental.pallas import tpu as pltpu
from jax.experimental.pallas import tpu_sc as plsc

NE = 10000
NR2 = 460
H = 256
CW = 16
NSUB = 16
EDG = 160000
K = 128
NCHUNK = EDG // K
SEG_PAD = 512
SEGT = SEG_PAD // 32
HALF = NE // 2
AGG_PAD = 5008
DEGT = 320

_f32 = jnp.float32
_i32 = jnp.int32


def _segsum_body(h2_hbm, rte_hbm, rseg_hbm, dst_hbm, out_sum, out_cnt, out_deg,
                 idx_v, seg_v, gid_v, a_v, acc_v, cnt_v, deg_v, gsem):
    cid = lax.axis_index("c")
    sid = lax.axis_index("s")
    wid = sid * 2 + cid
    zero = jnp.zeros((CW,), _f32)
    one = jnp.ones((CW,), _f32)
    cols = lax.iota(_i32, CW)

    def z1(r, c):
        acc_v[r, pl.ds(0, CW)] = zero
        return c
    lax.fori_loop(0, SEG_PAD, z1, None)

    def z2(r, c):
        cnt_v[r, pl.ds(0, CW)] = zero
        deg_v[r, pl.ds(0, CW)] = zero
        return c
    lax.fori_loop(0, DEGT + 16, z2, None)

    def chunk(ch, c):
        base = (cid * (NCHUNK // 2) + ch) * K
        pltpu.sync_copy(rte_hbm.at[pl.ds(base, K)], idx_v)
        pltpu.sync_copy(rseg_hbm.at[pl.ds(base, K)], seg_v)

        def gidx(j, c2):
            s = pl.ds(j * 16, 16)
            gid_v[s] = idx_v[s] * 16 + sid
            return c2
        lax.fori_loop(0, K // 16, gidx, None)
        pltpu.async_copy(h2_hbm.at[gid_v], a_v, gsem).wait()

        def row(r, c2):
            sp = jnp.full((CW,), r, _i32)
            segsp = plsc.load_gather(seg_v, [sp])
            plsc.addupdate_scatter(acc_v, [segsp, cols], a_v[r, pl.ds(0, CW)])
            return c2
        lax.fori_loop(0, K, row, None)
        return c
    lax.fori_loop(0, NCHUNK // 2, chunk, None)

    seg_lo = wid * SEGT

    def cchunk(ch, c):
        base = ch * K
        pltpu.sync_copy(rseg_hbm.at[pl.ds(base, K)], seg_v)

        def grp(j, c2):
            s = pl.ds(j * 16, 16)
            v = seg_v[s] - seg_lo
            m = (v >= 0) & (v < SEGT)
            loc = jnp.where(m, v, SEGT)
            plsc.addupdate_scatter(cnt_v, [loc, cols], one)
            return c2
        lax.fori_loop(0, K // 16, grp, None)
        return c
    lax.fori_loop(0, NCHUNK, cchunk, None)

    deg_lo = wid * DEGT

    def dchunk(ch, c):
        base = ch * K
        pltpu.sync_copy(dst_hbm.at[pl.ds(base, K)], seg_v)

        def grp(j, c2):
            s = pl.ds(j * 16, 16)
            v = seg_v[s] - deg_lo
            m = (v >= 0) & (v < DEGT)
            loc = jnp.where(m, v, DEGT)
            plsc.addupdate_scatter(deg_v, [loc, cols], one)
            return c2
        lax.fori_loop(0, K // 16, grp, None)
        return c
    lax.fori_loop(0, NCHUNK, dchunk, None)

    pltpu.sync_copy(acc_v.at[pl.ds(0, SEG_PAD)], out_sum.at[cid, sid])
    pltpu.sync_copy(cnt_v.at[pl.ds(0, SEGT)], out_cnt.at[wid])
    pltpu.sync_copy(deg_v.at[pl.ds(0, DEGT)], out_deg.at[wid])


@functools.lru_cache(maxsize=None)
def _segsum_kernel():
    mesh = plsc.VectorSubcoreMesh(core_axis_name="c", subcore_axis_name="s")
    return pl.kernel(
        _segsum_body,
        out_type=[jax.ShapeDtypeStruct((2, NSUB, SEG_PAD, CW), _f32),
                  jax.ShapeDtypeStruct((32, SEGT, CW), _f32),
                  jax.ShapeDtypeStruct((32, DEGT, CW), _f32)],
        mesh=mesh,
        scratch_types=[
            pltpu.VMEM((K,), _i32),
            pltpu.VMEM((K,), _i32),
            pltpu.VMEM((K,), _i32),
            pltpu.VMEM((K, CW), _f32),
            pltpu.VMEM((SEG_PAD + 16, CW), _f32),
            pltpu.VMEM((SEGT + 16, CW), _f32),
            pltpu.VMEM((DEGT + 16, CW), _f32),
            pltpu.SemaphoreType.DMA,
        ],
        compiler_params=pltpu.CompilerParams(needs_layout_passes=False, use_tc_tiling_on_sc=False),
    )


def _segsum(h2, rte, rseg, dst_t):
    return _segsum_kernel()(h2, rte, rseg, dst_t)


def _edge_body(hw2_hbm, h0w2_hbm, src_hbm, dst_hbm, et_hbm, out_agg,
               src_v, et_v, dst_v, ga_v, gb_v, a_v, b_v, acc_v, sem_a, sem_b):
    cid = lax.axis_index("c")
    sid = lax.axis_index("s")
    lo = cid * HALF
    zero = jnp.zeros((CW,), _f32)
    cols = lax.iota(_i32, CW)

    def z1(r, c):
        acc_v[r, pl.ds(0, CW)] = zero
        return c
    lax.fori_loop(0, HALF + 16, z1, None)

    def chunk(ch, c):
        base = ch * K
        pltpu.sync_copy(src_hbm.at[pl.ds(base, K)], src_v)
        pltpu.sync_copy(et_hbm.at[pl.ds(base, K)], et_v)
        pltpu.sync_copy(dst_hbm.at[pl.ds(base, K)], dst_v)

        def gidx(j, c2):
            s = pl.ds(j * 16, 16)
            ga_v[s] = src_v[s] * 16 + sid
            gb_v[s] = et_v[s] * 16 + sid
            v = dst_v[s] - lo
            m = (v >= 0) & (v < HALF)
            dst_v[s] = jnp.where(m, v, HALF)
            return c2
        lax.fori_loop(0, K // 16, gidx, None)
        ca = pltpu.async_copy(hw2_hbm.at[ga_v], a_v, sem_a)
        cb = pltpu.async_copy(h0w2_hbm.at[gb_v], b_v, sem_b)
        ca.wait()
        cb.wait()

        def row(r, c2):
            sp = jnp.full((CW,), r, _i32)
            locsp = plsc.load_gather(dst_v, [sp])
            x = a_v[r, pl.ds(0, CW)] + b_v[r, pl.ds(0, CW)]
            e = jnp.exp(x + x)
            y = 1.0 - 2.0 / (e + 1.0)
            plsc.addupdate_scatter(acc_v, [locsp, cols], y)
            return c2
        lax.fori_loop(0, K, row, None)
        return c
    lax.fori_loop(0, NCHUNK, chunk, None)

    pltpu.sync_copy(acc_v.at[pl.ds(0, AGG_PAD)], out_agg.at[cid, sid])


@functools.lru_cache(maxsize=None)
def _edge_kernel():
    mesh = plsc.VectorSubcoreMesh(core_axis_name="c", subcore_axis_name="s")
    return pl.kernel(
        _edge_body,
        out_type=jax.ShapeDtypeStruct((2, NSUB, AGG_PAD, CW), _f32),
        mesh=mesh,
        scratch_types=[
            pltpu.VMEM((K,), _i32),
            pltpu.VMEM((K,), _i32),
            pltpu.VMEM((K,), _i32),
            pltpu.VMEM((K,), _i32),
            pltpu.VMEM((K,), _i32),
            pltpu.VMEM((K, CW), _f32),
            pltpu.VMEM((K, CW), _f32),
            pltpu.VMEM((AGG_PAD + 16, CW), _f32),
            pltpu.SemaphoreType.DMA,
            pltpu.SemaphoreType.DMA,
        ],
        compiler_params=pltpu.CompilerParams(needs_layout_passes=False, use_tc_tiling_on_sc=False),
    )


def _edge(hw2, h0w2, src_t, dst_t, et_t):
    return _edge_kernel()(hw2, h0w2, src_t, dst_t, et_t)


def _l2n(x):
    n = jnp.sqrt(jnp.sum(x * x, axis=-1, keepdims=True))
    return x / jnp.maximum(n, 1e-12)


def _init_body(emb_ref, wc_ref, h_ref, hw_ref):
    h = _l2n(emb_ref[...])
    h_ref[...] = h
    hw_ref[...] = jnp.dot(h, wc_ref[...], preferred_element_type=_f32)


def _init_call(dynamic_emb, W_comp):
    blk = 1000
    return pl.pallas_call(
        _init_body,
        grid=(NE // blk,),
        in_specs=[pl.BlockSpec((blk, H), lambda i: (i, 0)),
                  pl.BlockSpec((H, H), lambda i: (0, 0))],
        out_specs=[pl.BlockSpec((blk, H), lambda i: (i, 0)),
                   pl.BlockSpec((blk, H), lambda i: (i, 0))],
        out_shape=[jax.ShapeDtypeStruct((NE, H), _f32),
                   jax.ShapeDtypeStruct((NE, H), _f32)],
    )(dynamic_emb, W_comp)


def _gibase_body(emb_ref, w_ref, b_ref, out_ref):
    out_ref[...] = (jnp.dot(emb_ref[...], w_ref[...], preferred_element_type=_f32)
                    + b_ref[...])


def _gibase_call(emb_rel_pad, W_top, b_ih):
    return pl.pallas_call(
        _gibase_body,
        out_shape=jax.ShapeDtypeStruct((SEG_PAD, 3 * H), _f32),
    )(emb_rel_pad, W_top, b_ih)


def _relgru_body(xs_ref, cnt_ref, gib_ref, wbot_ref, prev_ref, whh_ref,
                 bhh_ref, wc_ref, h0_ref, h0w_ref):
    xsum = xs_ref[0] + xs_ref[1]
    cnt = jnp.sum(cnt_ref[...], axis=-1, keepdims=True)
    xm = xsum / jnp.maximum(cnt, 1.0)
    prev = prev_ref[...]
    gi = gib_ref[...] + jnp.dot(xm, wbot_ref[...], preferred_element_type=_f32)
    gh = jnp.dot(prev, whh_ref[...], preferred_element_type=_f32) + bhh_ref[...]
    r = jax.nn.sigmoid(gi[:, :H] + gh[:, :H])
    z = jax.nn.sigmoid(gi[:, H:2 * H] + gh[:, H:2 * H])
    n = jnp.tanh(gi[:, 2 * H:] + r * gh[:, 2 * H:])
    h0 = _l2n((1.0 - z) * n + z * prev)
    h0_ref[...] = h0
    h0w_ref[...] = jnp.dot(h0, wc_ref[...], preferred_element_type=_f32)


def _relgru_call(xs, cnt, gi_base, wbot, prev, W_hh_rel, b_hh, W_comp):
    return pl.pallas_call(
        _relgru_body,
        out_shape=[jax.ShapeDtypeStruct((SEG_PAD, H), _f32),
                   jax.ShapeDtypeStruct((SEG_PAD, H), _f32)],
    )(xs, cnt, gi_base, wbot, prev, W_hh_rel, b_hh, W_comp)


def _entgru_body(agg_ref, deg_ref, h_ref, wih_ref, whh_ref, bih_ref, bhh_ref,
                 wc_ref, hn_ref, hwn_ref):
    agg = agg_ref[0]
    deg = jnp.sum(deg_ref[...], axis=-1, keepdims=True)
    cur = _l2n(agg / jnp.maximum(deg, 1.0))
    h = h_ref[...]
    gi = jnp.dot(cur, wih_ref[...], preferred_element_type=_f32) + bih_ref[...]
    gh = jnp.dot(h, whh_ref[...], preferred_element_type=_f32) + bhh_ref[...]
    r = jax.nn.sigmoid(gi[:, :H] + gh[:, :H])
    z = jax.nn.sigmoid(gi[:, H:2 * H] + gh[:, H:2 * H])
    n = jnp.tanh(gi[:, 2 * H:] + r * gh[:, 2 * H:])
    hn = _l2n((1.0 - z) * n + z * h)
    hn_ref[...] = hn
    hwn_ref[...] = jnp.dot(hn, wc_ref[...], preferred_element_type=_f32)


def _entgru_call(agg, deg, h, W_ih, W_hh, b_ih, b_hh, W_comp):
    blk = 1000
    nb = HALF // blk
    return pl.pallas_call(
        _entgru_body,
        grid=(NE // blk,),
        in_specs=[
            pl.BlockSpec((1, blk, H), lambda i: (i // nb, i % nb, 0)),
            pl.BlockSpec((blk, CW), lambda i: (i, 0)),
            pl.BlockSpec((blk, H), lambda i: (i, 0)),
            pl.BlockSpec((H, 3 * H), lambda i: (0, 0)),
            pl.BlockSpec((H, 3 * H), lambda i: (0, 0)),
            pl.BlockSpec((1, 3 * H), lambda i: (0, 0)),
            pl.BlockSpec((1, 3 * H), lambda i: (0, 0)),
            pl.BlockSpec((H, H), lambda i: (0, 0)),
        ],
        out_specs=[pl.BlockSpec((blk, H), lambda i: (i, 0)),
                   pl.BlockSpec((blk, H), lambda i: (i, 0))],
        out_shape=[jax.ShapeDtypeStruct((NE, H), _f32),
                   jax.ShapeDtypeStruct((NE, H), _f32)],
    )(agg, deg, h, W_ih, W_hh, b_ih, b_hh, W_comp)


def kernel(src, dst, etype, r_to_e, r_seg, dynamic_emb, emb_rel, W_comp,
           W_ih_rel, W_hh_rel, b_ih_rel, b_hh_rel,
           W_ih_rnn, W_hh_rnn, b_ih_rnn, b_hh_rnn):
    T = src.shape[0]
    emb_rel_pad = jnp.zeros((SEG_PAD, H), _f32).at[:NR2].set(emb_rel)
    h, hw = _init_call(dynamic_emb, W_comp)
    gi_base = _gibase_call(emb_rel_pad, W_ih_rel[:H], b_ih_rel.reshape(1, -1))
    wbot = W_ih_rel[H:]
    prev = emb_rel_pad
    hist = []
    for t in range(T):
        xs_p, cnt_p, deg_p = _segsum(h.reshape(NE * NSUB, CW), r_to_e[t],
                                     r_seg[t], dst[t])
        xs = xs_p.transpose(0, 2, 1, 3).reshape(2, SEG_PAD, H)
        cnt = cnt_p.reshape(SEG_PAD, CW)
        deg = deg_p.reshape(32 * DEGT, CW)[:NE]
        h0, h0w = _relgru_call(xs, cnt, gi_base, wbot, prev, W_hh_rel,
                               b_hh_rel.reshape(1, -1), W_comp)
        agg_p = _edge(hw.reshape(NE * NSUB, CW),
                      h0w.reshape(SEG_PAD * NSUB, CW), src[t], dst[t], etype[t])
        agg = agg_p.transpose(0, 2, 1, 3).reshape(2, AGG_PAD, H)
        h, hw = _entgru_call(agg, deg, h, W_ih_rnn, W_hh_rnn,
                             b_ih_rnn.reshape(1, -1), b_hh_rnn.reshape(1, -1),
                             W_comp)
        prev = h0
        hist.append(h)
    return jnp.stack(hist, axis=0), prev[:NR2]
